# baseline (speedup 1.0000x reference)
import functools

import jax
import jax.numpy as jnp
from jax import lax
from jax.experimental import pallas as pl
from jax.experimental.pallas import tpu as pltpu
from jax.experimental.pallas import tpu_sc as plsc


def _topk_body(s_ref, st_ref, idx_ref, val_ref):
    N = s_ref.shape[2]
    K = idx_ref.shape[2]
    scol = st_ref[0]
    rank = jnp.zeros((N, 1), jnp.float32)
    CJ = 512
    for c in range(N // CJ):
        srow_c = s_ref[0, :, c * CJ:(c + 1) * CJ]
        gt = srow_c > scol
        eq = srow_c == scol
        jidx = lax.broadcasted_iota(jnp.int32, (N, CJ), 1) + c * CJ
        iidx = lax.broadcasted_iota(jnp.int32, (N, CJ), 0)
        win = gt | (eq & (jidx < iidx))
        rank = rank + jnp.sum(jnp.where(win, 1.0, 0.0), axis=1, keepdims=True)
    iota_i = lax.broadcasted_iota(jnp.int32, (N, 1), 0).astype(jnp.float32)
    CR = 512
    for rc in range(K // CR):
        rr = (lax.broadcasted_iota(jnp.int32, (N, CR), 1)
              + rc * CR).astype(jnp.float32)
        M = jnp.where(rank == rr, 1.0, 0.0)
        val_ref[0, :, rc * CR:(rc + 1) * CR] = jnp.sum(
            M * scol, axis=0, keepdims=True)
        idxf = jnp.sum(M * iota_i, axis=0, keepdims=True)
        idx_ref[0, :, rc * CR:(rc + 1) * CR] = idxf.astype(jnp.int32)


def _topk(scores, K):
    B, N = scores.shape
    s3 = scores.reshape(B, 1, N)
    st = scores.reshape(B, N, 1)
    return pl.pallas_call(
        _topk_body,
        grid=(B,),
        in_specs=[
            pl.BlockSpec((1, 1, N), lambda b: (b, 0, 0)),
            pl.BlockSpec((1, N, 1), lambda b: (b, 0, 0)),
        ],
        out_specs=[
            pl.BlockSpec((1, 1, K), lambda b: (b, 0, 0)),
            pl.BlockSpec((1, 1, K), lambda b: (b, 0, 0)),
        ],
        out_shape=[
            jax.ShapeDtypeStruct((B, 1, K), jnp.int32),
            jax.ShapeDtypeStruct((B, 1, K), jnp.float32),
        ],
    )(s3, st)


_LO = 0x3F80
_HI = 0x3F800000


def _bin_body(g_ref, u_ref, ut_ref):
    x = g_ref[0]
    h2 = x.shape[1] // 2
    lo = jnp.where(x[:, :h2] != 0.0, _LO, 0)
    hi = jnp.where(x[:, h2:] != 0.0, _HI, 0)
    u_ref[0] = lo | hi
    xt = x.T
    lot = jnp.where(xt[:, :h2] != 0.0, _LO, 0)
    hit = jnp.where(xt[:, h2:] != 0.0, _HI, 0)
    ut_ref[0] = lot | hit


def _binarize_packed(g, TB=1024):
    B, N, _ = g.shape
    nb = N // TB
    return pl.pallas_call(
        _bin_body,
        grid=(B, nb, nb),
        in_specs=[
            pl.BlockSpec((1, TB, TB), lambda b, i, j: (b, i, j)),
        ],
        out_specs=[
            pl.BlockSpec((1, TB, TB // 2), lambda b, i, j: (b, i, j)),
            pl.BlockSpec((1, TB, TB // 2), lambda b, i, j: (b, j, i)),
        ],
        out_shape=[
            jax.ShapeDtypeStruct((B, N, N // 2), jnp.int32),
            jax.ShapeDtypeStruct((B, N, N // 2), jnp.int32),
        ],
    )(g)


def _sc_gather_h(h_flat, idx, B, A, N, F, K):
    C = 128
    NC = K // C
    mesh = plsc.VectorSubcoreMesh(core_axis_name="c", subcore_axis_name="s")

    @functools.partial(
        pl.kernel, mesh=mesh,
        out_type=jax.ShapeDtypeStruct((B * A * K, F), jnp.float32),
        scratch_types=[
            pltpu.VMEM((K,), jnp.int32),
            pltpu.VMEM((C, F), jnp.float32),
            pltpu.VMEM((C, F), jnp.float32),
            pltpu.SemaphoreType.DMA,
            pltpu.SemaphoreType.DMA,
            pltpu.SemaphoreType.DMA,
            pltpu.SemaphoreType.DMA,
        ],
    )
    def k_(h_hbm, idx_hbm, out_hbm, idx_v, buf0, buf1,
           gsem0, gsem1, ssem0, ssem1):
        wid = lax.axis_index("s") * 2 + lax.axis_index("c")
        b = wid // A
        a = wid % A
        pltpu.sync_copy(idx_hbm.at[b], idx_v)
        base = (b * A + a) * N

        def add_off(i, _):
            idx_v[pl.ds(i * 16, 16)] = idx_v[pl.ds(i * 16, 16)] + base
            return 0
        lax.fori_loop(0, K // 16, add_off, 0)

        bufs = (buf0, buf1)
        gsems = (gsem0, gsem1)
        ssems = (ssem0, ssem1)
        gets = [None, None]
        stores = [None, None]
        gets[0] = pltpu.async_copy(
            h_hbm.at[idx_v.at[pl.ds(0, C)]], bufs[0], gsems[0])
        for c in range(NC):
            bi = c & 1
            if c + 1 < NC:
                nbi = (c + 1) & 1
                if stores[nbi] is not None:
                    stores[nbi].wait()
                gets[nbi] = pltpu.async_copy(
                    h_hbm.at[idx_v.at[pl.ds((c + 1) * C, C)]],
                    bufs[nbi], gsems[nbi])
            gets[bi].wait()
            stores[bi] = pltpu.async_copy(
                bufs[bi], out_hbm.at[pl.ds(wid * K + c * C, C)], ssems[bi])
        for st in stores:
            if st is not None:
                st.wait()

    return k_(h_flat, idx)


def _sc_gather_u(u_flat, ut_flat, idx, B, N, K, W):
    NS = 8
    RPW = K // NS
    C2 = 32
    NC = RPW // C2
    dt = u_flat.dtype
    mesh = plsc.VectorSubcoreMesh(core_axis_name="c", subcore_axis_name="s")

    @functools.partial(
        pl.kernel, mesh=mesh,
        out_type=(jax.ShapeDtypeStruct((B * K, W), dt),
                  jax.ShapeDtypeStruct((B * K, W), dt)),
        scratch_types=[
            pltpu.VMEM((K,), jnp.int32),
            pltpu.VMEM((C2, W), dt),
            pltpu.VMEM((C2, W), dt),
            pltpu.SemaphoreType.DMA,
            pltpu.SemaphoreType.DMA,
            pltpu.SemaphoreType.DMA,
        ],
    )
    def k_(u_hbm, ut_hbm, idx_hbm, outr_hbm, outc_hbm,
           idx_v, buf0, buf1, gsem, ssem0, ssem1):
        wid = lax.axis_index("s") * 2 + lax.axis_index("c")
        b = wid // NS
        s8 = wid % NS
        pltpu.sync_copy(idx_hbm.at[b], idx_v)
        base = b * N

        def add_off(i, _):
            idx_v[pl.ds(i * 16, 16)] = idx_v[pl.ds(i * 16, 16)] + base
            return 0
        lax.fori_loop(0, K // 16, add_off, 0)

        bufs = (buf0, buf1)
        ssems = (ssem0, ssem1)
        stores = [None, None]
        jobs = [(u_hbm, outr_hbm, c) for c in range(NC)] + \
               [(ut_hbm, outc_hbm, c) for c in range(NC)]
        for j, (tbl, out, c) in enumerate(jobs):
            bi = j & 1
            off = s8 * RPW + c * C2
            if stores[bi] is not None:
                stores[bi].wait()
            pltpu.async_copy(
                tbl.at[idx_v.at[pl.ds(off, C2)]], bufs[bi], gsem).wait()
            stores[bi] = pltpu.async_copy(
                bufs[bi], out.at[pl.ds(b * K + off, C2)], ssems[bi])
        for st in stores:
            if st is not None:
                st.wait()

    return k_(u_flat, ut_flat, idx)


def _scale_body(x_ref, v_ref, o_ref):
    o_ref[0, 0] = x_ref[0, 0] * v_ref[0]


def _scale(raw, values_c):
    B, A, K, F = raw.shape
    return pl.pallas_call(
        _scale_body,
        grid=(B, A),
        in_specs=[
            pl.BlockSpec((1, 1, K, F), lambda b, a: (b, a, 0, 0)),
            pl.BlockSpec((1, K, 1), lambda b, a: (b, 0, 0)),
        ],
        out_specs=pl.BlockSpec((1, 1, K, F), lambda b, a: (b, a, 0, 0)),
        out_shape=jax.ShapeDtypeStruct((B, A, K, F), jnp.float32),
    )(raw, values_c)


def _unpack01(p):
    lo = jnp.where((p & 0xFFFF) != 0, 1.0, 0.0).astype(jnp.bfloat16)
    hi = jnp.where((p >> 16) != 0, 1.0, 0.0).astype(jnp.bfloat16)
    return jnp.concatenate([lo, hi], axis=1)


def _adj_body(r_ref, c_ref, o_ref):
    r2 = _unpack01(r_ref[0])
    c2 = _unpack01(c_ref[0])
    S = lax.dot_general(r2, c2, (((1,), (1,)), ((), ())),
                        preferred_element_type=jnp.float32)
    w = jnp.where(S > 0.0, 1.0, 0.0)
    deg = jnp.sum(w, axis=0, keepdims=True)
    o_ref[0] = w / deg


def _adj(R, Ct, JC=1):
    B, K, W2 = R.shape
    KJ = K // JC
    return pl.pallas_call(
        _adj_body,
        grid=(B, JC),
        in_specs=[
            pl.BlockSpec((1, K, W2), lambda b, j: (b, 0, 0)),
            pl.BlockSpec((1, KJ, W2), lambda b, j: (b, j, 0)),
        ],
        out_specs=pl.BlockSpec((1, K, KJ), lambda b, j: (b, 0, j)),
        out_shape=jax.ShapeDtypeStruct((B, K, K), jnp.float32),
    )(R, Ct)


def kernel(g, h, W, b):
    B, A, N, F = h.shape
    K = max(2, int(0.5 * N))

    weights = (h @ W.T + b)[..., 0]
    weights = jnp.transpose(weights, (0, 2, 1))
    weights = jnp.mean(weights, axis=-1)
    scores = jax.nn.sigmoid(weights)

    idx3, val3 = _topk(scores, K)
    idx = idx3.reshape(B, K)
    values = val3.reshape(B, K)

    raw = _sc_gather_h(h.reshape(B * A * N, F), idx, B, A, N, F, K)
    new_h = _scale(raw.reshape(B, A, K, F), values.reshape(B, K, 1))

    u, ut = _binarize_packed(g)
    Rf, Cf = _sc_gather_u(u.reshape(B * N, N // 2), ut.reshape(B * N, N // 2),
                          idx, B, N, K, N // 2)
    g_new = _adj(Rf.reshape(B, K, N // 2), Cf.reshape(B, K, N // 2))

    return (g_new, new_h, idx)


# speedup vs baseline: 1.0010x; 1.0010x over previous
import functools

import jax
import jax.numpy as jnp
from jax import lax
from jax.experimental import pallas as pl
from jax.experimental.pallas import tpu as pltpu
from jax.experimental.pallas import tpu_sc as plsc


def _topk_body(s_ref, st_ref, idx_ref, val_ref):
    N = s_ref.shape[2]
    K = idx_ref.shape[2]
    scol = st_ref[0]
    rank = jnp.zeros((N, 1), jnp.float32)
    CJ = 512
    for c in range(N // CJ):
        srow_c = s_ref[0, :, c * CJ:(c + 1) * CJ]
        gt = srow_c > scol
        eq = srow_c == scol
        jidx = lax.broadcasted_iota(jnp.int32, (N, CJ), 1) + c * CJ
        iidx = lax.broadcasted_iota(jnp.int32, (N, CJ), 0)
        win = gt | (eq & (jidx < iidx))
        rank = rank + jnp.sum(jnp.where(win, 1.0, 0.0), axis=1, keepdims=True)
    iota_i = lax.broadcasted_iota(jnp.int32, (N, 1), 0).astype(jnp.float32)
    CR = 512
    for rc in range(K // CR):
        rr = (lax.broadcasted_iota(jnp.int32, (N, CR), 1)
              + rc * CR).astype(jnp.float32)
        M = jnp.where(rank == rr, 1.0, 0.0)
        val_ref[0, :, rc * CR:(rc + 1) * CR] = jnp.sum(
            M * scol, axis=0, keepdims=True)
        idxf = jnp.sum(M * iota_i, axis=0, keepdims=True)
        idx_ref[0, :, rc * CR:(rc + 1) * CR] = idxf.astype(jnp.int32)


def _topk(scores, K):
    B, N = scores.shape
    s3 = scores.reshape(B, 1, N)
    st = scores.reshape(B, N, 1)
    return pl.pallas_call(
        _topk_body,
        grid=(B,),
        in_specs=[
            pl.BlockSpec((1, 1, N), lambda b: (b, 0, 0)),
            pl.BlockSpec((1, N, 1), lambda b: (b, 0, 0)),
        ],
        out_specs=[
            pl.BlockSpec((1, 1, K), lambda b: (b, 0, 0)),
            pl.BlockSpec((1, 1, K), lambda b: (b, 0, 0)),
        ],
        out_shape=[
            jax.ShapeDtypeStruct((B, 1, K), jnp.int32),
            jax.ShapeDtypeStruct((B, 1, K), jnp.float32),
        ],
    )(s3, st)


_LO = 0x3F80
_HI = 0x3F800000


def _bin_body(g_ref, u_ref, ut_ref):
    x = g_ref[0]
    h2 = x.shape[1] // 2
    lo = jnp.where(x[:, :h2] != 0.0, _LO, 0)
    hi = jnp.where(x[:, h2:] != 0.0, _HI, 0)
    u_ref[0] = lo | hi
    xt = x.T
    lot = jnp.where(xt[:, :h2] != 0.0, _LO, 0)
    hit = jnp.where(xt[:, h2:] != 0.0, _HI, 0)
    ut_ref[0] = lot | hit


def _binarize_packed(g, TB=1024):
    B, N, _ = g.shape
    nb = N // TB
    return pl.pallas_call(
        _bin_body,
        grid=(B, nb, nb),
        in_specs=[
            pl.BlockSpec((1, TB, TB), lambda b, i, j: (b, i, j)),
        ],
        out_specs=[
            pl.BlockSpec((1, TB, TB // 2), lambda b, i, j: (b, i, j)),
            pl.BlockSpec((1, TB, TB // 2), lambda b, i, j: (b, j, i)),
        ],
        out_shape=[
            jax.ShapeDtypeStruct((B, N, N // 2), jnp.int32),
            jax.ShapeDtypeStruct((B, N, N // 2), jnp.int32),
        ],
    )(g)


def _sc_gather_h(h_flat, idx, B, A, N, F, K):
    C = 128
    NC = K // C
    mesh = plsc.VectorSubcoreMesh(core_axis_name="c", subcore_axis_name="s")

    @functools.partial(
        pl.kernel, mesh=mesh,
        out_type=jax.ShapeDtypeStruct((B * A * K, F), jnp.float32),
        scratch_types=[
            pltpu.VMEM((K,), jnp.int32),
            pltpu.VMEM((C, F), jnp.float32),
            pltpu.VMEM((C, F), jnp.float32),
            pltpu.SemaphoreType.DMA,
            pltpu.SemaphoreType.DMA,
            pltpu.SemaphoreType.DMA,
            pltpu.SemaphoreType.DMA,
        ],
    )
    def k_(h_hbm, idx_hbm, out_hbm, idx_v, buf0, buf1,
           gsem0, gsem1, ssem0, ssem1):
        wid = lax.axis_index("s") * 2 + lax.axis_index("c")
        b = wid // A
        a = wid % A
        pltpu.sync_copy(idx_hbm.at[b], idx_v)
        base = (b * A + a) * N

        def add_off(i, _):
            idx_v[pl.ds(i * 16, 16)] = idx_v[pl.ds(i * 16, 16)] + base
            return 0
        lax.fori_loop(0, K // 16, add_off, 0)

        bufs = (buf0, buf1)
        gsems = (gsem0, gsem1)
        ssems = (ssem0, ssem1)
        gets = [None, None]
        stores = [None, None]
        gets[0] = pltpu.async_copy(
            h_hbm.at[idx_v.at[pl.ds(0, C)]], bufs[0], gsems[0])
        for c in range(NC):
            bi = c & 1
            if c + 1 < NC:
                nbi = (c + 1) & 1
                if stores[nbi] is not None:
                    stores[nbi].wait()
                gets[nbi] = pltpu.async_copy(
                    h_hbm.at[idx_v.at[pl.ds((c + 1) * C, C)]],
                    bufs[nbi], gsems[nbi])
            gets[bi].wait()
            stores[bi] = pltpu.async_copy(
                bufs[bi], out_hbm.at[pl.ds(wid * K + c * C, C)], ssems[bi])
        for st in stores:
            if st is not None:
                st.wait()

    return k_(h_flat, idx)


def _sc_gather_u(u_flat, ut_flat, idx, B, N, K, W):
    NS = 8
    RPW = K // NS
    C2 = 32
    NC = RPW // C2
    dt = u_flat.dtype
    mesh = plsc.VectorSubcoreMesh(core_axis_name="c", subcore_axis_name="s")

    @functools.partial(
        pl.kernel, mesh=mesh,
        out_type=(jax.ShapeDtypeStruct((B * K, W), dt),
                  jax.ShapeDtypeStruct((B * K, W), dt)),
        scratch_types=[
            pltpu.VMEM((K,), jnp.int32),
            pltpu.VMEM((C2, W), dt),
            pltpu.VMEM((C2, W), dt),
            pltpu.SemaphoreType.DMA,
            pltpu.SemaphoreType.DMA,
            pltpu.SemaphoreType.DMA,
        ],
    )
    def k_(u_hbm, ut_hbm, idx_hbm, outr_hbm, outc_hbm,
           idx_v, buf0, buf1, gsem, ssem0, ssem1):
        wid = lax.axis_index("s") * 2 + lax.axis_index("c")
        b = wid // NS
        s8 = wid % NS
        pltpu.sync_copy(idx_hbm.at[b], idx_v)
        base = b * N

        def add_off(i, _):
            idx_v[pl.ds(i * 16, 16)] = idx_v[pl.ds(i * 16, 16)] + base
            return 0
        lax.fori_loop(0, K // 16, add_off, 0)

        bufs = (buf0, buf1)
        ssems = (ssem0, ssem1)
        stores = [None, None]
        jobs = [(u_hbm, outr_hbm, c) for c in range(NC)] + \
               [(ut_hbm, outc_hbm, c) for c in range(NC)]
        for j, (tbl, out, c) in enumerate(jobs):
            bi = j & 1
            off = s8 * RPW + c * C2
            if stores[bi] is not None:
                stores[bi].wait()
            pltpu.async_copy(
                tbl.at[idx_v.at[pl.ds(off, C2)]], bufs[bi], gsem).wait()
            stores[bi] = pltpu.async_copy(
                bufs[bi], out.at[pl.ds(b * K + off, C2)], ssems[bi])
        for st in stores:
            if st is not None:
                st.wait()

    return k_(u_flat, ut_flat, idx)


def _scale_body(x_ref, v_ref, o_ref):
    o_ref[0, 0] = x_ref[0, 0] * v_ref[0]


def _scale(raw, values_c):
    B, A, K, F = raw.shape
    return pl.pallas_call(
        _scale_body,
        grid=(B, A),
        in_specs=[
            pl.BlockSpec((1, 1, K, F), lambda b, a: (b, a, 0, 0)),
            pl.BlockSpec((1, K, 1), lambda b, a: (b, 0, 0)),
        ],
        out_specs=pl.BlockSpec((1, 1, K, F), lambda b, a: (b, a, 0, 0)),
        out_shape=jax.ShapeDtypeStruct((B, A, K, F), jnp.float32),
    )(raw, values_c)


def _unpack01(p):
    lo = jnp.where((p & 0xFFFF) != 0, 1.0, 0.0).astype(jnp.bfloat16)
    hi = jnp.where((p >> 16) != 0, 1.0, 0.0).astype(jnp.bfloat16)
    return jnp.concatenate([lo, hi], axis=1)


def _adj_body(r_ref, c_ref, o_ref):
    r2 = _unpack01(r_ref[0])
    c2 = _unpack01(c_ref[0])
    S = lax.dot_general(r2, c2, (((1,), (1,)), ((), ())),
                        preferred_element_type=jnp.float32)
    w = jnp.where(S > 0.0, 1.0, 0.0)
    deg = jnp.sum(w, axis=0, keepdims=True)
    o_ref[0] = w / deg


def _adj(R, Ct, JC=2):
    B, K, W2 = R.shape
    KJ = K // JC
    return pl.pallas_call(
        _adj_body,
        grid=(B, JC),
        in_specs=[
            pl.BlockSpec((1, K, W2), lambda b, j: (b, 0, 0)),
            pl.BlockSpec((1, KJ, W2), lambda b, j: (b, j, 0)),
        ],
        out_specs=pl.BlockSpec((1, K, KJ), lambda b, j: (b, 0, j)),
        out_shape=jax.ShapeDtypeStruct((B, K, K), jnp.float32),
    )(R, Ct)


def kernel(g, h, W, b):
    B, A, N, F = h.shape
    K = max(2, int(0.5 * N))

    weights = (h @ W.T + b)[..., 0]
    weights = jnp.transpose(weights, (0, 2, 1))
    weights = jnp.mean(weights, axis=-1)
    scores = jax.nn.sigmoid(weights)

    idx3, val3 = _topk(scores, K)
    idx = idx3.reshape(B, K)
    values = val3.reshape(B, K)

    raw = _sc_gather_h(h.reshape(B * A * N, F), idx, B, A, N, F, K)
    new_h = _scale(raw.reshape(B, A, K, F), values.reshape(B, K, 1))

    u, ut = _binarize_packed(g)
    Rf, Cf = _sc_gather_u(u.reshape(B * N, N // 2), ut.reshape(B * N, N // 2),
                          idx, B, N, K, N // 2)
    g_new = _adj(Rf.reshape(B, K, N // 2), Cf.reshape(B, K, N // 2))

    return (g_new, new_h, idx)
